# baseline (speedup 1.0000x reference)
import jax
import jax.numpy as jnp
from jax.experimental import pallas as pl

_BB = 8
_NS = 4
_HB = _BB // _NS
_RS = 256


def _matvec_body(d0_ref, d1_ref, d2_ref, d3_ref, c_ref, o_ref):
    gb = pl.program_id(0) * _BB
    cv = c_ref[pl.ds(gb, _BB), :]
    for j, d_ref in enumerate((d0_ref, d1_ref, d2_ref, d3_ref)):
        for i in range(_HB):
            bi = j * _HB + i
            d = d_ref[i]
            w = cv[bi:bi + 1, 0:1024]
            acc = jnp.sum(d * w, axis=1)
            o_ref[bi, 0] = acc + cv[bi, 1024]


def kernel(dict, coef):
    D = dict
    B, L, K = D.shape
    KC = coef.shape[1]

    dspec = [
        pl.BlockSpec((_HB, _RS, K),
                     (lambda s: (lambda b, r: (_NS * b + s, r, 0)))(s))
        for s in range(_NS)
    ]
    out = pl.pallas_call(
        _matvec_body,
        grid=(B // _BB, L // _RS),
        in_specs=dspec + [pl.BlockSpec((B, KC), lambda b, r: (0, 0))],
        out_specs=pl.BlockSpec((_BB, 1, _RS), lambda b, r: (b, 0, r)),
        out_shape=jax.ShapeDtypeStruct((B, 1, L), jnp.float32),
    )(D, D, D, D, coef)
    return out.reshape(B, L, 1)
